# baseline (speedup 1.0000x reference)
import functools

import jax
import jax.numpy as jnp
from jax import lax
from jax.experimental import pallas as pl
from jax.experimental.pallas import tpu as pltpu
from jax.experimental.pallas import tpu_sc as plsc

_M = 4096
_N = 5000
_NUM_CLASSES = 600
_IN_SIZE = 256 * 7 * 7
_REP = 1024

_TM = 2048
_TF = 256
_TK = 896
_NM = _M // _TM
_NK = _IN_SIZE // _TK

def _pair_gather(scores, labels, h_idx, o_idx):
    info = plsc.get_sparse_core_info()
    nc, ns, lanes = info.num_cores, info.num_subcores, info.num_lanes
    chunk = _M // (nc * ns)

    def body(scores_hbm, labels_hbm, hidx_hbm, oidx_hbm,
             prod_hbm, cls_hbm,
             scores_v, labels_v, hidx_v, oidx_v, prod_v, cls_v):
        wid = lax.axis_index("s") * nc + lax.axis_index("c")
        base = wid * chunk
        pltpu.sync_copy(scores_hbm, scores_v)
        pltpu.sync_copy(labels_hbm, labels_v)
        pltpu.sync_copy(hidx_hbm.at[pl.ds(base, chunk)], hidx_v)
        pltpu.sync_copy(oidx_hbm.at[pl.ds(base, chunk)], oidx_v)
        for j in range(chunk // lanes):
            sl = pl.ds(j * lanes, lanes)
            hi = hidx_v[sl]
            oi = oidx_v[sl]
            s_h = plsc.load_gather(scores_v, [hi])
            s_o = plsc.load_gather(scores_v, [oi])
            lab = plsc.load_gather(labels_v, [oi])
            prod_v[sl] = s_h * s_o
            cls_v[sl] = lab
        pltpu.sync_copy(prod_v, prod_hbm.at[pl.ds(base, chunk)])
        pltpu.sync_copy(cls_v, cls_hbm.at[pl.ds(base, chunk)])

    fn = functools.partial(
        pl.kernel,
        mesh=plsc.VectorSubcoreMesh(core_axis_name="c", subcore_axis_name="s"),
        compiler_params=pltpu.CompilerParams(needs_layout_passes=False),
        out_type=[jax.ShapeDtypeStruct((_M,), jnp.float32),
                  jax.ShapeDtypeStruct((_M,), jnp.int32)],
        scratch_types=[
            pltpu.VMEM((_N,), jnp.float32),
            pltpu.VMEM((_N,), jnp.int32),
            pltpu.VMEM((chunk,), jnp.int32),
            pltpu.VMEM((chunk,), jnp.int32),
            pltpu.VMEM((chunk,), jnp.float32),
            pltpu.VMEM((chunk,), jnp.int32),
        ],
    )(body)
    return fn(scores, labels, h_idx, o_idx)


def _mlp_body(prod_ref, cls_ref, x_ref, w1_ref, b1_ref, w2_ref, b2_ref,
              w3_ref, b3_ref, out_ref, acc_ref):
    k = pl.program_id(1)

    xb = x_ref[...].astype(jnp.bfloat16)
    wb = w1_ref[...].astype(jnp.bfloat16)
    partial = jnp.dot(xb, wb, preferred_element_type=jnp.float32)

    @pl.when(k == 0)
    def _first():
        acc_ref[...] = partial

    @pl.when(k > 0)
    def _accum():
        acc_ref[...] += partial

    @pl.when(k == _NK - 1)
    def _finalize():
        w2b = w2_ref[...].astype(jnp.bfloat16)
        w3b = w3_ref[...].astype(jnp.bfloat16)
        cols = lax.broadcasted_iota(jnp.int32, (_TF, _NUM_CLASSES), 1)
        for c in range(_TM // _TF):
            rows = pl.ds(c * _TF, _TF)
            h1 = jnp.maximum(acc_ref[rows, :] + b1_ref[...], 0.0)
            h1 = h1.astype(jnp.bfloat16)
            h2 = jnp.maximum(
                jnp.dot(h1, w2b, preferred_element_type=jnp.float32)
                + b2_ref[...], 0.0).astype(jnp.bfloat16)
            logits = (jnp.dot(h2, w3b, preferred_element_type=jnp.float32)
                      + b3_ref[...])
            p = jax.nn.sigmoid(logits)
            out_ref[rows, :] = jnp.where(
                cols == cls_ref[rows, :], p * prod_ref[rows, :], 0.0)


def _mlp_call(prod2, cls2, x, W1, b1r, W2, b2r, W3, b3r):
    return pl.pallas_call(
        _mlp_body,
        grid=(_NM, _NK),
        in_specs=[
            pl.BlockSpec((_TM, 1), lambda m, k: (m, 0)),
            pl.BlockSpec((_TM, 1), lambda m, k: (m, 0)),
            pl.BlockSpec((_TM, _TK), lambda m, k: (m, k)),
            pl.BlockSpec((_TK, _REP), lambda m, k: (k, 0)),
            pl.BlockSpec((1, _REP), lambda m, k: (0, 0)),
            pl.BlockSpec((_REP, _REP), lambda m, k: (0, 0)),
            pl.BlockSpec((1, _REP), lambda m, k: (0, 0)),
            pl.BlockSpec((_REP, _NUM_CLASSES), lambda m, k: (0, 0)),
            pl.BlockSpec((1, _NUM_CLASSES), lambda m, k: (0, 0)),
        ],
        out_specs=pl.BlockSpec((_TM, _NUM_CLASSES), lambda m, k: (m, 0)),
        out_shape=jax.ShapeDtypeStruct((_M, _NUM_CLASSES), jnp.float32),
        scratch_shapes=[pltpu.VMEM((_TM, _REP), jnp.float32)],
        compiler_params=pltpu.CompilerParams(
            dimension_semantics=("arbitrary", "arbitrary"),
            vmem_limit_bytes=64 * 1024 * 1024,
        ),
    )(prod2, cls2, x, W1, b1r, W2, b2r, W3, b3r)


def kernel(pooled_features, scores, labels, paired_idx, W1, b1, W2, b2, W3, b3):
    h_idx = paired_idx[:, 0].astype(jnp.int32)
    o_idx = paired_idx[:, 1].astype(jnp.int32)
    prod, cls = _pair_gather(scores, labels.astype(jnp.int32), h_idx, o_idx)
    return _mlp_call(
        prod.reshape(_M, 1), cls.reshape(_M, 1), pooled_features,
        W1, b1.reshape(1, _REP), W2, b2.reshape(1, _REP),
        W3, b3.reshape(1, _NUM_CLASSES))
